# baseline (speedup 1.0000x reference)
import functools

import jax
import jax.numpy as jnp
import numpy as np
from jax import lax
from jax.experimental import pallas as pl
from jax.experimental.pallas import tpu as pltpu
from jax.experimental.pallas import tpu_sc as plsc

RADIUS = 0.1
DILATION = 0.05
NDIM = 3
KS = 3
IN_CH = 64
OUT_CH = 64
TI = 128
PW = 128
CW = 896
NB = 256
NWORK = 32


def _cell_offsets():
    g = (np.arange(KS) - (KS - 1) / 2.0) * DILATION
    mesh = np.stack(np.meshgrid(*([g] * NDIM), indexing="ij"), axis=-1)
    return mesh.reshape(-1, NDIM)


_OFFS = _cell_offsets()

_SC_MESH = plsc.VectorSubcoreMesh(core_axis_name="c", subcore_axis_name="s")


def _worker_id():
    return lax.axis_index("s") * 2 + lax.axis_index("c")


def _rank_kernel(xc_ref, xr_ref, rank_ref, binstart_ref):
    t = pl.program_id(0)
    n = xr_ref.shape[1]
    xi = xc_ref[:]
    xj = xr_ref[:]
    jj = lax.broadcasted_iota(jnp.int32, (1, n), 1)
    ii = TI * t + lax.broadcasted_iota(jnp.int32, (TI, 1), 0)
    before = (xj < xi) | ((xj == xi) & (jj < ii))
    rank_ref[:] = jnp.sum(before.astype(jnp.int32), axis=1, keepdims=True)

    nbt = NB // (2048 // TI)
    binj = (xr_ref[:] * float(NB)).astype(jnp.int32)
    bb = nbt * t + lax.broadcasted_iota(jnp.int32, (nbt, 1), 0)
    binstart_ref[:] = jnp.sum((binj < bb).astype(jnp.int32), axis=1,
                              keepdims=True)


def _cols_kernel(rankr_ref, xr_ref, yr_ref, zr_ref, denr_ref,
                 xs_ref, ys_ref, zs_ref, dens_ref):
    t = pl.program_id(0)
    n = rankr_ref.shape[1]
    rr = rankr_ref[:]
    r0 = TI * t + lax.broadcasted_iota(jnp.int32, (TI, 1), 0)
    oh = (rr == r0).astype(jnp.float32)
    for src, dst in ((xr_ref, xs_ref), (yr_ref, ys_ref),
                     (zr_ref, zs_ref), (denr_ref, dens_ref)):
        dst[:] = jnp.sum(oh * src[:], axis=1, keepdims=True)


def _scatter_body(n, rank_hbm, data_hbm, datas_hbm, idxv, rows, sem):
    ch = n // NWORK
    base = _worker_id() * ch
    pltpu.sync_copy(rank_hbm.at[pl.ds(base, ch)], idxv)
    pltpu.sync_copy(data_hbm.at[pl.ds(base, ch)], rows)
    pltpu.async_copy(rows, datas_hbm.at[idxv], sem).wait()


def _conv_kernel(locs_tile_ref, locs_t_ref, data_ref, den_ref, wkt_ref, b_ref,
                 rankc_ref, binstart_ref, out_ref, ds_ref, outs_ref):
    n = locs_t_ref.shape[1]
    nt = n // TI
    t = pl.program_id(0)

    @pl.when(t == 0)
    def _():
        ds_ref[:] = data_ref[:, :IN_CH] * (1.0 / den_ref[:])

    @pl.when(t < nt)
    def _():
        li = locs_tile_ref[:]
        lx, ly, lz = li[:, 0:1], li[:, 1:2], li[:, 2:3]
        xmin = jnp.min(lx)
        xmax = jnp.max(lx)

        inv_h = 1.0 / RADIUS
        ih2 = inv_h * inv_h
        acc = jnp.zeros((TI, OUT_CH), dtype=jnp.float32)
        for s in (float(-DILATION), 0.0, float(DILATION)):
            a = xmin + (s - RADIUS)
            bidx = (a * float(NB) + 1024.0).astype(jnp.int32) - 1024
            bidx = jnp.minimum(jnp.maximum(bidx, 0), NB - 1)
            lo = binstart_ref[bidx]
            lo = (lo // 128) * 128
            lo = jnp.minimum(lo, n - CW)
            hbin = ((xmax + (s + RADIUS)) * float(NB)).astype(jnp.int32) + 1
            hi = jnp.where(hbin >= NB, n,
                           binstart_ref[jnp.minimum(hbin, NB - 1)])
            nch = jnp.minimum((hi - lo + TI - 1) // TI, CW // TI)

            ks = [k for k in range(_OFFS.shape[0])
                  if float(_OFFS[k][0]) == s]

            def chunk_body(cc, fks, s=s, lo=lo, ks=ks, lx=lx, ly=ly, lz=lz):
                co = lo + cc * TI
                jxw = locs_t_ref[0:1, pl.ds(co, TI)]
                jyw = locs_t_ref[1:2, pl.ds(co, TI)]
                jzw = locs_t_ref[2:3, pl.ds(co, TI)]
                dxw = lx - jxw
                dyw = ly - jyw
                dzw = lz - jzw
                d2s = ((dxw * dxw + dyw * dyw + dzw * dzw)
                       + (2.0 * s) * dxw) * ih2
                dsw = ds_ref[pl.ds(co, TI), :]
                ty = dyw * (2.0 * DILATION * ih2)
                tz = dzw * (2.0 * DILATION * ih2)
                base = {}
                for ay, az in ((0.0, 0.0), (0.0, DILATION),
                               (DILATION, 0.0), (DILATION, DILATION)):
                    c = (s * s + ay * ay + az * az) * ih2 + 1e-10
                    base[(ay, az)] = d2s + c
                out = []
                for j, k in enumerate(ks):
                    _, oy, oz = (float(v) for v in _OFFS[k])
                    q = base[(abs(oy), abs(oz))]
                    if oy > 0.0:
                        q = q + ty
                    elif oy < 0.0:
                        q = q - ty
                    if oz > 0.0:
                        q = q + tz
                    elif oz < 0.0:
                        q = q - tz
                    q = jnp.maximum(q, 1e-10)
                    u = jnp.maximum(1.0 - q * lax.rsqrt(q), 0.0)
                    w = u * u * u
                    out.append(fks[j] + jnp.dot(
                        w, dsw, preferred_element_type=jnp.float32))
                return tuple(out)

            fks = lax.fori_loop(
                0, nch, chunk_body,
                tuple(jnp.zeros((TI, IN_CH), jnp.float32) for _ in ks))
            for j, k in enumerate(ks):
                acc = acc + jnp.dot(fks[j], wkt_ref[k],
                                    preferred_element_type=jnp.float32)

        outs_ref[pl.ds(t * TI, TI), :] = acc + b_ref[:]

    @pl.when(t >= nt)
    def _():
        ri = rankc_ref[:]
        jj = lax.broadcasted_iota(jnp.int32, (1, n), 1)
        oh = (ri == jj).astype(jnp.float32)
        out_ref[:] = jnp.dot(oh, outs_ref[:],
                             preferred_element_type=jnp.float32)


@jax.jit
def kernel(locs, data, density, W, b):
    B, n, _ = locs.shape
    ch = n // NWORK
    nt = n // TI
    locs2 = locs.reshape(n, NDIM)
    x = locs2[:, 0]
    y = locs2[:, 1]
    z = locs2[:, 2]
    den = density.reshape(n)
    data2 = data.reshape(n, IN_CH)
    xc = x.reshape(n, 1)
    xr = x.reshape(1, n)

    nbt = NB // nt
    rank2, binstart2 = pl.pallas_call(
        _rank_kernel,
        grid=(nt,),
        in_specs=[
            pl.BlockSpec((TI, 1), lambda i: (i, 0)),
            pl.BlockSpec((1, n), lambda i: (0, 0)),
        ],
        out_specs=[
            pl.BlockSpec((TI, 1), lambda i: (i, 0)),
            pl.BlockSpec((nbt, 1), lambda i: (i, 0)),
        ],
        out_shape=[
            jax.ShapeDtypeStruct((n, 1), jnp.int32),
            jax.ShapeDtypeStruct((NB, 1), jnp.int32),
        ],
    )(xc, xr)
    rank = rank2.reshape(n)
    binstart = binstart2.reshape(NB)

    xs, ys, zs, dens = pl.pallas_call(
        _cols_kernel,
        grid=(nt,),
        in_specs=[pl.BlockSpec((1, n), lambda i: (0, 0))] * 5,
        out_specs=[pl.BlockSpec((TI, 1), lambda i: (i, 0))] * 4,
        out_shape=[jax.ShapeDtypeStruct((n, 1), jnp.float32)] * 4,
    )(rank.reshape(1, n), xr, y.reshape(1, n), z.reshape(1, n),
      den.reshape(1, n))

    data_p = jnp.concatenate(
        [data2, jnp.zeros((n, PW - IN_CH), jnp.float32)], axis=1)
    scatterk = pl.kernel(
        functools.partial(_scatter_body, n),
        out_type=jax.ShapeDtypeStruct((n, PW), jnp.float32),
        mesh=_SC_MESH,
        scratch_types=[
            pltpu.VMEM((ch,), jnp.int32),
            pltpu.VMEM((ch, PW), jnp.float32),
            pltpu.SemaphoreType.DMA,
        ],
    )
    data_s = scatterk(rank, data_p)

    locs_s = jnp.concatenate([xs, ys, zs], axis=1)
    locs_t = locs_s.T
    den_s = dens
    wkt = jnp.transpose(W, (2, 1, 0))
    b2 = b.reshape(1, OUT_CH)

    out = pl.pallas_call(
        _conv_kernel,
        grid=(2 * nt,),
        in_specs=[
            pl.BlockSpec((TI, NDIM), lambda i: (jnp.minimum(i, nt - 1), 0)),
            pl.BlockSpec((NDIM, n), lambda i: (0, 0)),
            pl.BlockSpec((n, PW), lambda i: (0, 0)),
            pl.BlockSpec((n, 1), lambda i: (0, 0)),
            pl.BlockSpec((_OFFS.shape[0], IN_CH, OUT_CH), lambda i: (0, 0, 0)),
            pl.BlockSpec((1, OUT_CH), lambda i: (0, 0)),
            pl.BlockSpec((TI, 1), lambda i: (jnp.maximum(i - nt, 0), 0)),
            pl.BlockSpec(memory_space=pltpu.SMEM),
        ],
        out_specs=pl.BlockSpec((TI, OUT_CH),
                               lambda i: (jnp.maximum(i - nt, 0), 0)),
        out_shape=jax.ShapeDtypeStruct((n, OUT_CH), jnp.float32),
        scratch_shapes=[
            pltpu.VMEM((n, IN_CH), jnp.float32),
            pltpu.VMEM((n, OUT_CH), jnp.float32),
        ],
    )(locs_s, locs_t, data_s, den_s, wkt, b2, rank2, binstart)

    return out.reshape(B, n, OUT_CH)


# speedup vs baseline: 1.3806x; 1.3806x over previous
import functools

import jax
import jax.numpy as jnp
import numpy as np
from jax import lax
from jax.experimental import pallas as pl
from jax.experimental.pallas import tpu as pltpu
from jax.experimental.pallas import tpu_sc as plsc

RADIUS = 0.1
DILATION = 0.05
NDIM = 3
KS = 3
IN_CH = 64
OUT_CH = 64
TI = 128
PW = 128
CW = 896
NB = 256
NWORK = 32


def _cell_offsets():
    g = (np.arange(KS) - (KS - 1) / 2.0) * DILATION
    mesh = np.stack(np.meshgrid(*([g] * NDIM), indexing="ij"), axis=-1)
    return mesh.reshape(-1, NDIM)


_OFFS = _cell_offsets()

_SC_MESH = plsc.VectorSubcoreMesh(core_axis_name="c", subcore_axis_name="s")


def _worker_id():
    return lax.axis_index("s") * 2 + lax.axis_index("c")


def _rank_kernel(xc_ref, xr_ref, rank_ref, binstart_ref):
    t = pl.program_id(0)
    n = xr_ref.shape[1]
    xi = xc_ref[:]
    xj = xr_ref[:]
    jj = lax.broadcasted_iota(jnp.int32, (1, n), 1)
    ii = TI * t + lax.broadcasted_iota(jnp.int32, (TI, 1), 0)
    before = (xj < xi) | ((xj == xi) & (jj < ii))
    rank_ref[:] = jnp.sum(before.astype(jnp.int32), axis=1, keepdims=True)

    nbt = NB // (2048 // TI)
    binj = (xr_ref[:] * float(NB)).astype(jnp.int32)
    bb = nbt * t + lax.broadcasted_iota(jnp.int32, (nbt, 1), 0)
    binstart_ref[:] = jnp.sum((binj < bb).astype(jnp.int32), axis=1,
                              keepdims=True)


def _cols_kernel(rankr_ref, xr_ref, yr_ref, zr_ref, denr_ref,
                 xs_ref, ys_ref, zs_ref, dens_ref):
    t = pl.program_id(0)
    n = rankr_ref.shape[1]
    rr = rankr_ref[:]
    r0 = TI * t + lax.broadcasted_iota(jnp.int32, (TI, 1), 0)
    oh = (rr == r0).astype(jnp.float32)
    for src, dst in ((xr_ref, xs_ref), (yr_ref, ys_ref),
                     (zr_ref, zs_ref), (denr_ref, dens_ref)):
        dst[:] = jnp.sum(oh * src[:], axis=1, keepdims=True)


def _scatter_body(n, rank_hbm, data_hbm, datas_hbm, idxv, rows, sem):
    ch = n // NWORK
    base = _worker_id() * ch
    pltpu.sync_copy(rank_hbm.at[pl.ds(base, ch)], idxv)
    pltpu.sync_copy(data_hbm.at[pl.ds(base, ch)], rows)
    pltpu.async_copy(rows, datas_hbm.at[idxv], sem).wait()


def _conv_kernel(locs_tile_ref, locs_t_ref, data_ref, den_ref, wkt_ref, b_ref,
                 rankc_ref, binstart_ref, out_ref, ds_ref, outs_ref):
    n = locs_t_ref.shape[1]
    nt = n // TI
    t = pl.program_id(0)

    @pl.when(t == 0)
    def _():
        ds_ref[:] = data_ref[:, :IN_CH] * (1.0 / den_ref[:])

    @pl.when(t < nt)
    def _():
        li = locs_tile_ref[:]
        lx, ly, lz = li[:, 0:1], li[:, 1:2], li[:, 2:3]
        xmin = jnp.min(lx)
        xmax = jnp.max(lx)

        inv_h = 1.0 / RADIUS
        ih2 = inv_h * inv_h
        acc = jnp.zeros((TI, OUT_CH), dtype=jnp.float32)
        for s in (float(-DILATION), 0.0, float(DILATION)):
            a = xmin + (s - RADIUS)
            bidx = (a * float(NB) + 1024.0).astype(jnp.int32) - 1024
            bidx = jnp.minimum(jnp.maximum(bidx, 0), NB - 1)
            lo = binstart_ref[bidx]
            lo = (lo // 128) * 128
            lo = jnp.minimum(lo, n - CW)
            ks = [k for k in range(_OFFS.shape[0])
                  if float(_OFFS[k][0]) == s]

            def chunk_body(cc, fks, s=s, lo=lo, ks=ks, lx=lx, ly=ly, lz=lz):
                co = lo + cc * TI
                jxw = locs_t_ref[0:1, pl.ds(co, TI)]
                jyw = locs_t_ref[1:2, pl.ds(co, TI)]
                jzw = locs_t_ref[2:3, pl.ds(co, TI)]
                dxw = lx - jxw
                dyw = ly - jyw
                dzw = lz - jzw
                d2s = ((dxw * dxw + dyw * dyw + dzw * dzw)
                       + (2.0 * s) * dxw) * ih2
                dsw = ds_ref[pl.ds(co, TI), :]
                ty = dyw * (2.0 * DILATION * ih2)
                tz = dzw * (2.0 * DILATION * ih2)
                base = {}
                for ay, az in ((0.0, 0.0), (0.0, DILATION),
                               (DILATION, 0.0), (DILATION, DILATION)):
                    c = (s * s + ay * ay + az * az) * ih2 + 1e-10
                    base[(ay, az)] = d2s + c
                out = []
                for j, k in enumerate(ks):
                    _, oy, oz = (float(v) for v in _OFFS[k])
                    q = base[(abs(oy), abs(oz))]
                    if oy > 0.0:
                        q = q + ty
                    elif oy < 0.0:
                        q = q - ty
                    if oz > 0.0:
                        q = q + tz
                    elif oz < 0.0:
                        q = q - tz
                    q = jnp.maximum(q, 1e-10)
                    u = jnp.maximum(1.0 - q * lax.rsqrt(q), 0.0)
                    w = u * u * u
                    out.append(fks[j] + jnp.dot(
                        w, dsw, preferred_element_type=jnp.float32))
                return tuple(out)

            fks = tuple(jnp.zeros((TI, IN_CH), jnp.float32) for _ in ks)
            for cc in range(CW // TI):
                fks = chunk_body(cc, fks)
            for j, k in enumerate(ks):
                acc = acc + jnp.dot(fks[j], wkt_ref[k],
                                    preferred_element_type=jnp.float32)

        outs_ref[pl.ds(t * TI, TI), :] = acc + b_ref[:]

    @pl.when(t >= nt)
    def _():
        ri = rankc_ref[:]
        jj = lax.broadcasted_iota(jnp.int32, (1, n), 1)
        oh = (ri == jj).astype(jnp.float32)
        out_ref[:] = jnp.dot(oh, outs_ref[:],
                             preferred_element_type=jnp.float32)


@jax.jit
def kernel(locs, data, density, W, b):
    B, n, _ = locs.shape
    ch = n // NWORK
    nt = n // TI
    locs2 = locs.reshape(n, NDIM)
    x = locs2[:, 0]
    y = locs2[:, 1]
    z = locs2[:, 2]
    den = density.reshape(n)
    data2 = data.reshape(n, IN_CH)
    xc = x.reshape(n, 1)
    xr = x.reshape(1, n)

    nbt = NB // nt
    rank2, binstart2 = pl.pallas_call(
        _rank_kernel,
        grid=(nt,),
        in_specs=[
            pl.BlockSpec((TI, 1), lambda i: (i, 0)),
            pl.BlockSpec((1, n), lambda i: (0, 0)),
        ],
        out_specs=[
            pl.BlockSpec((TI, 1), lambda i: (i, 0)),
            pl.BlockSpec((nbt, 1), lambda i: (i, 0)),
        ],
        out_shape=[
            jax.ShapeDtypeStruct((n, 1), jnp.int32),
            jax.ShapeDtypeStruct((NB, 1), jnp.int32),
        ],
    )(xc, xr)
    rank = rank2.reshape(n)
    binstart = binstart2.reshape(NB)

    xs, ys, zs, dens = pl.pallas_call(
        _cols_kernel,
        grid=(nt,),
        in_specs=[pl.BlockSpec((1, n), lambda i: (0, 0))] * 5,
        out_specs=[pl.BlockSpec((TI, 1), lambda i: (i, 0))] * 4,
        out_shape=[jax.ShapeDtypeStruct((n, 1), jnp.float32)] * 4,
    )(rank.reshape(1, n), xr, y.reshape(1, n), z.reshape(1, n),
      den.reshape(1, n))

    data_p = jnp.concatenate(
        [data2, jnp.zeros((n, PW - IN_CH), jnp.float32)], axis=1)
    scatterk = pl.kernel(
        functools.partial(_scatter_body, n),
        out_type=jax.ShapeDtypeStruct((n, PW), jnp.float32),
        mesh=_SC_MESH,
        scratch_types=[
            pltpu.VMEM((ch,), jnp.int32),
            pltpu.VMEM((ch, PW), jnp.float32),
            pltpu.SemaphoreType.DMA,
        ],
    )
    data_s = scatterk(rank, data_p)

    locs_s = jnp.concatenate([xs, ys, zs], axis=1)
    locs_t = locs_s.T
    den_s = dens
    wkt = jnp.transpose(W, (2, 1, 0))
    b2 = b.reshape(1, OUT_CH)

    out = pl.pallas_call(
        _conv_kernel,
        grid=(2 * nt,),
        in_specs=[
            pl.BlockSpec((TI, NDIM), lambda i: (jnp.minimum(i, nt - 1), 0)),
            pl.BlockSpec((NDIM, n), lambda i: (0, 0)),
            pl.BlockSpec((n, PW), lambda i: (0, 0)),
            pl.BlockSpec((n, 1), lambda i: (0, 0)),
            pl.BlockSpec((_OFFS.shape[0], IN_CH, OUT_CH), lambda i: (0, 0, 0)),
            pl.BlockSpec((1, OUT_CH), lambda i: (0, 0)),
            pl.BlockSpec((TI, 1), lambda i: (jnp.maximum(i - nt, 0), 0)),
            pl.BlockSpec(memory_space=pltpu.SMEM),
        ],
        out_specs=pl.BlockSpec((TI, OUT_CH),
                               lambda i: (jnp.maximum(i - nt, 0), 0)),
        out_shape=jax.ShapeDtypeStruct((n, OUT_CH), jnp.float32),
        scratch_shapes=[
            pltpu.VMEM((n, IN_CH), jnp.float32),
            pltpu.VMEM((n, OUT_CH), jnp.float32),
        ],
    )(locs_s, locs_t, data_s, den_s, wkt, b2, rank2, binstart)

    return out.reshape(B, n, OUT_CH)
